# baseline (speedup 1.0000x reference)
import functools
import jax
import jax.numpy as jnp
from jax import lax
from jax.experimental import pallas as pl
from jax.experimental.pallas import tpu as pltpu
from jax.experimental.pallas import tpu_sc as plsc

MAX_POS = 4096
HIDDEN = 1024

info = plsc.get_sparse_core_info()
NC, NS = info.num_cores, info.num_subcores
NW = NC * NS
CHUNK = 64
SC_B = 2


def _sc_body(table_hbm, out_hbm, buf):
    wid = lax.axis_index("s") * NC + lax.axis_index("c")
    rows_per_w = MAX_POS // NW
    base = wid * rows_per_w
    for c in range(rows_per_w // CHUNK):
        r = base + c * CHUNK
        pltpu.sync_copy(table_hbm.at[pl.ds(r, CHUNK), :], buf)
        for b in range(SC_B):
            pltpu.sync_copy(buf, out_hbm.at[b, pl.ds(r, CHUNK), :])


def _tc_body(emb_ref, out_ref):
    out_ref[...] = jnp.broadcast_to(emb_ref[...][None], out_ref.shape)


def kernel(position_ids, position_embeddings):
    B, S, H = position_ids.shape
    table = position_embeddings[:S]
    tc_b = B - SC_B

    mesh = plsc.VectorSubcoreMesh(core_axis_name="c", subcore_axis_name="s")
    sc_k = functools.partial(
        pl.kernel,
        mesh=mesh,
        out_type=jax.ShapeDtypeStruct((SC_B, S, H), jnp.float32),
        scratch_types=[pltpu.VMEM((CHUNK, H), jnp.float32)],
    )(_sc_body)
    sc_out = sc_k(table)

    block_s = 512
    tc_out = pl.pallas_call(
        _tc_body,
        grid=(S // block_s,),
        in_specs=[pl.BlockSpec((block_s, H), lambda i: (i, 0))],
        out_specs=pl.BlockSpec((tc_b, block_s, H), lambda i: (0, i, 0)),
        out_shape=jax.ShapeDtypeStruct((tc_b, S, H), jnp.float32),
    )(table)

    return jnp.concatenate([tc_out, sc_out], axis=0)


# speedup vs baseline: 1.9877x; 1.9877x over previous
import functools
import jax
import jax.numpy as jnp
from jax import lax
from jax.experimental import pallas as pl
from jax.experimental.pallas import tpu as pltpu
from jax.experimental.pallas import tpu_sc as plsc

MAX_POS = 4096
HIDDEN = 1024
B = 4

info = plsc.get_sparse_core_info()
NC, NS = info.num_cores, info.num_subcores
NW = NC * NS
ROWS_PER_W = MAX_POS // NW
CHUNK = 32
NCHUNK = ROWS_PER_W // CHUNK
NBUF = 2


def _sc_body(table_hbm, out_hbm, buf0, buf1, rsem0, rsem1, wsem0, wsem1):
    wid = lax.axis_index("s") * NC + lax.axis_index("c")
    base = wid * ROWS_PER_W
    bufs = (buf0, buf1)
    rsems = (rsem0, rsem1)
    wsems = (wsem0, wsem1)

    reads = [None] * NCHUNK
    writes = [[] for _ in range(NCHUNK)]

    def start_read(c):
        r = base + c * CHUNK
        reads[c] = pltpu.make_async_copy(
            table_hbm.at[pl.ds(r, CHUNK), :], bufs[c % NBUF], rsems[c % NBUF]
        )
        reads[c].start()

    def start_writes(c):
        r = base + c * CHUNK
        for b in range(B):
            d = pltpu.make_async_copy(
                bufs[c % NBUF], out_hbm.at[b, pl.ds(r, CHUNK), :], wsems[c % NBUF]
            )
            d.start()
            writes[c].append(d)

    start_read(0)
    if NCHUNK > 1:
        start_read(1)
    for c in range(NCHUNK):
        reads[c].wait()
        start_writes(c)
        nxt = c + NBUF
        if nxt < NCHUNK:
            for d in writes[c]:
                d.wait()
            start_read(nxt)
        elif c >= NCHUNK - NBUF:
            pass
    for c in range(max(0, NCHUNK - NBUF), NCHUNK):
        for d in writes[c]:
            d.wait()


def kernel(position_ids, position_embeddings):
    Bd, S, H = position_ids.shape
    mesh = plsc.VectorSubcoreMesh(core_axis_name="c", subcore_axis_name="s")
    k = functools.partial(
        pl.kernel,
        mesh=mesh,
        out_type=jax.ShapeDtypeStruct((Bd, S, H), jnp.float32),
        scratch_types=[
            pltpu.VMEM((CHUNK, HIDDEN), jnp.float32),
            pltpu.VMEM((CHUNK, HIDDEN), jnp.float32),
            pltpu.SemaphoreType.DMA,
            pltpu.SemaphoreType.DMA,
            pltpu.SemaphoreType.DMA,
            pltpu.SemaphoreType.DMA,
        ],
    )(_sc_body)
    return k(position_embeddings[:S])


# speedup vs baseline: 3.5089x; 1.7653x over previous
import jax
import jax.numpy as jnp
from jax.experimental import pallas as pl
from jax.experimental.pallas import tpu as pltpu

BLOCK_S = 1024


def _bcast_body(emb_ref, out_ref):
    out_ref[...] = jnp.broadcast_to(emb_ref[...][None], out_ref.shape)


def kernel(position_ids, position_embeddings):
    B, S, H = position_ids.shape
    out = pl.pallas_call(
        _bcast_body,
        grid=(S // BLOCK_S,),
        in_specs=[pl.BlockSpec((BLOCK_S, H), lambda i: (i, 0))],
        out_specs=pl.BlockSpec((B, BLOCK_S, H), lambda i: (0, i, 0)),
        out_shape=jax.ShapeDtypeStruct((B, S, H), jnp.float32),
        compiler_params=pltpu.CompilerParams(
            dimension_semantics=("arbitrary",),
        ),
    )(position_embeddings[:S])
    return out
